# baseline (speedup 1.0000x reference)
import functools

import jax
import jax.numpy as jnp
from jax import lax
from jax.experimental import pallas as pl
from jax.experimental.pallas import tpu as pltpu
from jax.experimental.pallas import tpu_sc as plsc

L = 16
NC = 2
NS = 16
NW = NC * NS
IDX_CHUNK = 128


def _make_sc_gather(batch, seq, dim, vocab, chunk, chunks_per_row):
    mod = vocab - 1
    n_idx_chunks = chunk // IDX_CHUNK
    n_vecs = chunk // L
    dim_blocks = dim // 8
    st_per_chunk = chunk // IDX_CHUNK
    n_st = seq // IDX_CHUNK

    @functools.partial(
        pl.kernel,
        out_type=jax.ShapeDtypeStruct((batch, dim_blocks, n_st, 8, IDX_CHUNK),
                                      jnp.float32),
        mesh=plsc.VectorSubcoreMesh(core_axis_name="c", subcore_axis_name="s",
                                    num_cores=NC, num_subcores=NS),
        scratch_types=[
            pltpu.VMEM((chunk + 8,), jnp.int32),
            pltpu.VMEM((n_idx_chunks, IDX_CHUNK), jnp.int32),
            pltpu.VMEM((dim_blocks, st_per_chunk, 8, IDX_CHUNK), jnp.float32),
            pltpu.VMEM((L,), jnp.float32),
            pltpu.SemaphoreType.DMA,
        ],
        compiler_params=pltpu.CompilerParams(use_tc_tiling_on_sc=False),
    )
    def body(ids_hbm, scale_hbm, tab_hbm, out_hbm,
             buf, idx2, rows, sv_ref, sem):
        cid = lax.axis_index("c")
        sid = lax.axis_index("s")
        wid = sid * NC + cid
        base = wid * chunk
        b_idx = wid // chunks_per_row
        st0 = (wid % chunks_per_row) * st_per_chunk
        row_pos = lax.rem(wid, chunks_per_row)
        at_row_start = row_pos == 0

        pltpu.sync_copy(scale_hbm, sv_ref)

        @pl.when(at_row_start)
        def _():
            pltpu.sync_copy(ids_hbm.at[pl.ds(base, chunk)],
                            buf.at[pl.ds(8, chunk)])

        @pl.when(jnp.logical_not(at_row_start))
        def _():
            pltpu.sync_copy(ids_hbm.at[pl.ds(base - 8, chunk + 8)], buf)

        lane = lax.iota(jnp.int32, L)
        row_pos_vec = jnp.full((L,), row_pos, jnp.int32)
        first_key = lane + row_pos_vec * jnp.int32(64)
        for k in range(n_vecs):
            cur = buf[pl.ds(8 + k * L, L)]
            prev = buf[pl.ds(7 + k * L, L)]
            h = jnp.mod((cur * jnp.int32(36313)) ^ (prev * jnp.int32(27191)),
                        jnp.int32(mod))
            if k == 0:
                h = jnp.where(first_key == 0, jnp.int32(mod), h)
            idx2[k * L // IDX_CHUNK, pl.ds((k * L) % IDX_CHUNK, L)] = h

        def fire(d):
            dt = d >> 3
            dsub = d & 7
            copies = []
            for c in range(n_idx_chunks):
                copies.append(pltpu.async_copy(
                    tab_hbm.at[d].at[idx2.at[c]],
                    rows.at[dt].at[c].at[dsub],
                    sem,
                ))
            return copies

        def drain(copies):
            for cp in copies:
                cp.wait()

        prev = fire(jnp.int32(0))

        def gather_dim(d, carry):
            copies = fire(d)
            drain(copies)
            return carry

        lax.fori_loop(1, dim, gather_dim, 0)
        drain(prev)

        sv = sv_ref[...]

        def scale_row(rr, carry):
            dt = rr >> 5
            c = (rr >> 3) & 3
            dsub = rr & 7
            for k in range(IDX_CHUNK // L):
                rows[dt, c, dsub, pl.ds(k * L, L)] = (
                    rows[dt, c, dsub, pl.ds(k * L, L)] * sv)
            return carry

        lax.fori_loop(0, dim_blocks * st_per_chunk * 8, scale_row, 0)

        pltpu.sync_copy(rows, out_hbm.at[b_idx].at[:, pl.ds(st0, st_per_chunk)])

    return body


def kernel(ids, embed_weight, scale):
    b, s = ids.shape
    vocab, dim = embed_weight.shape
    total = b * s
    chunk = total // NW
    chunks_per_row = s // chunk
    ids_flat = ids.reshape(total)
    scale_vec = jnp.broadcast_to(scale.astype(jnp.float32), (L,))
    tab_t = embed_weight.T
    fn = _make_sc_gather(b, s, dim, vocab, chunk, chunks_per_row)
    out5 = fn(ids_flat, scale_vec, tab_t)
    return out5.transpose(0, 2, 4, 1, 3).reshape(b, s, dim)


# speedup vs baseline: 7.6697x; 7.6697x over previous
import functools

import jax
import jax.numpy as jnp
from jax import lax
from jax.experimental import pallas as pl
from jax.experimental.pallas import tpu as pltpu
from jax.experimental.pallas import tpu_sc as plsc

L = 16
NC = 2
NS = 16
NW = NC * NS
BLK = 128


def _make_sc_gather(batch, seq, dim, vocab, chunk, chunks_per_row):
    mod = vocab - 1
    n_vecs = chunk // L
    n_blk = chunk // BLK
    dim_blocks = dim // 8
    n_st = seq // BLK

    @functools.partial(
        pl.kernel,
        out_type=jax.ShapeDtypeStruct((batch, dim_blocks, n_st, 8, BLK),
                                      jnp.float32),
        mesh=plsc.VectorSubcoreMesh(core_axis_name="c", subcore_axis_name="s",
                                    num_cores=NC, num_subcores=NS),
        scratch_types=[
            pltpu.VMEM((chunk + 8,), jnp.int32),
            pltpu.VMEM((n_blk, BLK), jnp.int32),
            pltpu.VMEM((n_blk, BLK), jnp.int32),
            pltpu.VMEM((2, BLK, 2 * dim), jnp.float32),
            pltpu.VMEM((dim_blocks, n_blk, 8, BLK), jnp.float32),
            pltpu.VMEM((L,), jnp.float32),
            pltpu.SemaphoreType.DMA,
        ],
        compiler_params=pltpu.CompilerParams(use_tc_tiling_on_sc=True,
                                             needs_layout_passes=False),
    )
    def body(ids_hbm, scale_hbm, tab_hbm, out_hbm,
             buf, lidx, half, stage, rows, sv_ref, sem):
        cid = lax.axis_index("c")
        sid = lax.axis_index("s")
        wid = sid * NC + cid
        base = wid * chunk
        b_idx = wid // chunks_per_row
        st0 = (wid % chunks_per_row) * n_blk
        row_pos = lax.rem(wid, chunks_per_row)
        at_row_start = row_pos == 0

        pltpu.sync_copy(scale_hbm, sv_ref)

        @pl.when(at_row_start)
        def _():
            pltpu.sync_copy(ids_hbm.at[pl.ds(base, chunk)],
                            buf.at[pl.ds(8, chunk)])

        @pl.when(jnp.logical_not(at_row_start))
        def _():
            pltpu.sync_copy(ids_hbm.at[pl.ds(base - 8, chunk + 8)], buf)

        lane = lax.iota(jnp.int32, L)
        row_pos_vec = jnp.full((L,), row_pos, jnp.int32)
        first_key = lane + row_pos_vec * jnp.int32(64)
        for k in range(n_vecs):
            cur = buf[pl.ds(8 + k * L, L)]
            prev = buf[pl.ds(7 + k * L, L)]
            h = jnp.mod((cur * jnp.int32(36313)) ^ (prev * jnp.int32(27191)),
                        jnp.int32(mod))
            if k == 0:
                h = jnp.where(first_key == 0, jnp.int32(mod), h)
            cc = k * L // BLK
            off = (k * L) % BLK
            lidx[cc, pl.ds(off, L)] = h >> 1
            half[cc, pl.ds(off, L)] = (h & jnp.int32(1)) * jnp.int32(dim)

        sv = sv_ref[...]

        def fire(cc):
            return pltpu.async_copy(
                tab_hbm.at[lidx.at[cc]],
                stage.at[cc & 1],
                sem,
            )

        def extract(cc, cp):
            cp.wait()
            par = cc & 1

            def per_dim(d, carry):
                dt = d >> 3
                dsub = d & 7
                dvec = jnp.full((L,), d, jnp.int32)
                for idv in range(BLK // L):
                    ivec = lane + idv * L
                    colv = half[cc, pl.ds(idv * L, L)] + dvec
                    v = plsc.load_gather(stage.at[par], [ivec, colv])
                    rows[dt, cc, dsub, pl.ds(idv * L, L)] = v * sv
                return carry

            lax.fori_loop(0, dim, per_dim, 0)

        prev = fire(jnp.int32(0))

        def blk_loop(cc, carry):
            cp = fire(cc)
            extract(cc - 1, cp)
            return carry

        lax.fori_loop(1, n_blk, blk_loop, 0)
        extract(jnp.int32(n_blk - 1), prev)

        pltpu.sync_copy(rows, out_hbm.at[b_idx].at[:, pl.ds(st0, n_blk)])

    return body


def kernel(ids, embed_weight, scale):
    b, s = ids.shape
    vocab, dim = embed_weight.shape
    total = b * s
    chunk = total // NW
    chunks_per_row = s // chunk
    ids_flat = ids.reshape(total)
    scale_vec = jnp.broadcast_to(scale.astype(jnp.float32), (L,))
    tab2 = embed_weight.reshape(vocab // 2, 2 * dim)
    fn = _make_sc_gather(b, s, dim, vocab, chunk, chunks_per_row)
    out5 = fn(ids_flat, scale_vec, tab2)
    return out5.transpose(0, 2, 4, 1, 3).reshape(b, s, dim)


# speedup vs baseline: 7.9674x; 1.0388x over previous
import functools

import jax
import jax.numpy as jnp
from jax import lax
from jax.experimental import pallas as pl
from jax.experimental.pallas import tpu as pltpu
from jax.experimental.pallas import tpu_sc as plsc

L = 16
NC = 2
NS = 16
NW = NC * NS
IDX_CHUNK = 128


def _make_sc_gather(total, dim, vocab, chunk, chunks_per_row):
    mod = vocab - 1
    n_gathers = chunk // IDX_CHUNK
    n_vecs = chunk // L

    @functools.partial(
        pl.kernel,
        out_type=jax.ShapeDtypeStruct((total, dim), jnp.float32),
        mesh=plsc.VectorSubcoreMesh(core_axis_name="c", subcore_axis_name="s",
                                    num_cores=NC, num_subcores=NS),
        scratch_types=[
            pltpu.VMEM((chunk + 8,), jnp.int32),
            pltpu.VMEM((n_gathers, IDX_CHUNK), jnp.int32),
            pltpu.VMEM((chunk, dim), jnp.float32),
            pltpu.VMEM((L,), jnp.float32),
            pltpu.SemaphoreType.DMA,
        ],
        compiler_params=pltpu.CompilerParams(use_tc_tiling_on_sc=False),
    )
    def body(ids_hbm, scale_hbm, table_hbm, out_hbm, buf, idx2, rows, sv_ref, sem):
        cid = lax.axis_index("c")
        sid = lax.axis_index("s")
        wid = sid * NC + cid
        base = wid * chunk
        row_pos = lax.rem(wid, chunks_per_row)
        at_row_start = row_pos == 0

        pltpu.sync_copy(scale_hbm, sv_ref)

        @pl.when(at_row_start)
        def _():
            pltpu.sync_copy(ids_hbm.at[pl.ds(base, chunk)],
                            buf.at[pl.ds(8, chunk)])

        @pl.when(jnp.logical_not(at_row_start))
        def _():
            pltpu.sync_copy(ids_hbm.at[pl.ds(base - 8, chunk + 8)], buf)

        lane = lax.iota(jnp.int32, L)
        row_pos_vec = jnp.full((L,), row_pos, jnp.int32)
        first_key = lane + row_pos_vec * jnp.int32(64)
        for k in range(n_vecs):
            cur = buf[pl.ds(8 + k * L, L)]
            prev = buf[pl.ds(7 + k * L, L)]
            h = jnp.mod((cur * jnp.int32(36313)) ^ (prev * jnp.int32(27191)),
                        jnp.int32(mod))
            if k == 0:
                h = jnp.where(first_key == 0, jnp.int32(mod), h)
            idx2[k * L // IDX_CHUNK, pl.ds((k * L) % IDX_CHUNK, L)] = h

        copies = []
        for g in range(n_gathers):
            copies.append(pltpu.async_copy(
                table_hbm.at[idx2.at[g]],
                rows.at[pl.ds(g * IDX_CHUNK, IDX_CHUNK)],
                sem,
            ))
        for cp in copies:
            cp.wait()

        sv = sv_ref[...]

        def scale_row(i, carry):
            for cc in range(dim // L):
                rows[i, pl.ds(cc * L, L)] = rows[i, pl.ds(cc * L, L)] * sv
            return carry

        lax.fori_loop(0, chunk, scale_row, 0)

        pltpu.sync_copy(rows, out_hbm.at[pl.ds(base, chunk)])

    return body


def kernel(ids, embed_weight, scale):
    b, s = ids.shape
    vocab, dim = embed_weight.shape
    total = b * s
    chunk = total // NW
    chunks_per_row = s // chunk
    ids_flat = ids.reshape(total)
    scale_vec = jnp.broadcast_to(scale.astype(jnp.float32), (L,))
    fn = _make_sc_gather(total, dim, vocab, chunk, chunks_per_row)
    out = fn(ids_flat, scale_vec, embed_weight)
    return out.reshape(b, s, dim)
